# baseline (speedup 1.0000x reference)
import functools

import jax
import jax.numpy as jnp
from jax import lax
from jax.experimental import pallas as pl
from jax.experimental.pallas import tpu as pltpu
from jax.experimental.pallas import tpu_sc as plsc

_LANES = 16


def _gather_mean_sc(ids, table):
    n_items, hist = ids.shape
    _, d = table.shape
    info = plsc.get_sparse_core_info()
    nw = info.num_cores * info.num_subcores
    ipw = n_items // nw
    nvec = d // _LANES
    mesh = plsc.VectorSubcoreMesh(core_axis_name="c", subcore_axis_name="s")

    @functools.partial(
        pl.kernel,
        out_type=jax.ShapeDtypeStruct((n_items, d), jnp.float32),
        mesh=mesh,
        scratch_types=[
            pltpu.VMEM((ipw, hist), jnp.int32),
            pltpu.VMEM((hist, d), jnp.float32),
            pltpu.VMEM((hist, d), jnp.float32),
            pltpu.VMEM((ipw, d), jnp.float32),
            pltpu.SemaphoreType.DMA,
            pltpu.SemaphoreType.DMA,
        ],
        compiler_params=pltpu.CompilerParams(use_tc_tiling_on_sc=False),
    )
    def body(ids_hbm, table_hbm, out_hbm, idx_v, buf0, buf1, acc_v, sem0, sem1):
        wid = lax.axis_index("s") * info.num_cores + lax.axis_index("c")
        base = wid * ipw
        pltpu.sync_copy(ids_hbm.at[pl.ds(base, ipw)], idx_v)
        pltpu.async_copy(table_hbm.at[idx_v.at[0]], buf0, sem0)
        pltpu.async_copy(table_hbm.at[idx_v.at[1]], buf1, sem1)

        scale = jnp.float32(1.0 / hist)

        def reduce_into(buf, row):
            def rbody(r, accs):
                return tuple(
                    accs[j] + buf[r, pl.ds(_LANES * j, _LANES)] for j in range(nvec)
                )
            accs = lax.fori_loop(
                0, hist, rbody,
                tuple(jnp.zeros((_LANES,), jnp.float32) for _ in range(nvec)),
            )
            for j in range(nvec):
                acc_v[row, pl.ds(_LANES * j, _LANES)] = accs[j] * scale

        def pair_body(k, carry):
            i = k * 2
            for b, (buf, sem) in enumerate(((buf0, sem0), (buf1, sem1))):
                it = i + b
                pltpu.make_async_copy(table_hbm.at[idx_v.at[it]], buf, sem).wait()
                reduce_into(buf, it)
                nxt = it + 2

                @pl.when(nxt < ipw)
                def _():
                    pltpu.async_copy(table_hbm.at[idx_v.at[nxt]], buf, sem)
            return carry

        lax.fori_loop(0, ipw // 2, pair_body, 0)
        pltpu.sync_copy(acc_v, out_hbm.at[pl.ds(base, ipw)])

    return body(ids, table)


def _mm_body(r_ref, t_ref, o_ref):
    o_ref[...] = jnp.dot(r_ref[...], t_ref[...], preferred_element_type=jnp.float32)


def _matmul_tc(ratio, pooled):
    b, n = ratio.shape
    _, d = pooled.shape
    bb = 256
    return pl.pallas_call(
        _mm_body,
        grid=(b // bb,),
        in_specs=[
            pl.BlockSpec((bb, n), lambda i: (i, 0)),
            pl.BlockSpec((n, d), lambda i: (0, 0)),
        ],
        out_specs=pl.BlockSpec((bb, d), lambda i: (i, 0)),
        out_shape=jax.ShapeDtypeStruct((b, d), jnp.float32),
    )(ratio, pooled)


def kernel(input_ids, input_ratio, embedding):
    ids = input_ids.astype(jnp.int32)
    pooled = _gather_mean_sc(ids, embedding)
    return _matmul_tc(input_ratio, pooled)


# speedup vs baseline: 1.0264x; 1.0264x over previous
import functools

import jax
import jax.numpy as jnp
from jax import lax
from jax.experimental import pallas as pl
from jax.experimental.pallas import tpu as pltpu
from jax.experimental.pallas import tpu_sc as plsc

_LANES = 16
_CHUNK_ITEMS = 2
_NSLOTS = 4


def _gather_sum_sc(ids2, tgt2, table, n_items, hist):
    _, d = table.shape
    clen = _CHUNK_ITEMS * hist
    info = plsc.get_sparse_core_info()
    nc, ns = info.num_cores, info.num_subcores
    nw = nc * ns
    ipw = n_items // nw
    rpw = ipw // _CHUNK_ITEMS
    nvec = d // _LANES
    mesh = plsc.VectorSubcoreMesh(core_axis_name="c", subcore_axis_name="s")

    @functools.partial(
        pl.kernel,
        out_type=jax.ShapeDtypeStruct((n_items, d), jnp.float32),
        mesh=mesh,
        scratch_types=[
            pltpu.VMEM((rpw, clen), jnp.int32),
            pltpu.VMEM((rpw, clen), jnp.int32),
            pltpu.VMEM((_NSLOTS, clen, d), jnp.float32),
            pltpu.VMEM((ipw, d), jnp.float32),
            pltpu.VMEM_SHARED((ns * ipw, d), jnp.float32),
            pltpu.SemaphoreType.DMA,
            pltpu.SemaphoreType.DMA,
            pltpu.SemaphoreType.DMA,
            pltpu.SemaphoreType.DMA,
            pltpu.SemaphoreType.DMA,
            pltpu.SemaphoreType.DMA,
            pltpu.SemaphoreType.DMA,
            pltpu.SemaphoreType.DMA,
        ],
        compiler_params=pltpu.CompilerParams(use_tc_tiling_on_sc=False),
    )
    def body(ids_hbm, tgt_hbm, table_hbm, out_hbm, idx_v, tgt_v, buf, zeros_v,
             acc_s, *sems):
        sem_g, sem_s = sems[:_NSLOTS], sems[_NSLOTS:]
        sid = lax.axis_index("s")
        wid = sid * nc + lax.axis_index("c")
        pltpu.sync_copy(ids_hbm.at[pl.ds(wid * rpw, rpw)], idx_v)
        pltpu.sync_copy(tgt_hbm.at[sid], tgt_v)

        zeros = jnp.zeros((_LANES,), jnp.float32)

        def zbody(i, c):
            for j in range(nvec):
                zeros_v[i, pl.ds(_LANES * j, _LANES)] = zeros
            return c

        lax.fori_loop(0, ipw, zbody, 0)
        pltpu.sync_copy(zeros_v, acc_s.at[pl.ds(sid * ipw, ipw)])

        for c in range(2):
            pltpu.async_copy(table_hbm.at[idx_v.at[c]], buf.at[c], sem_g[c])

        def steps(kk, carry):
            for b in range(_NSLOTS):
                k = kk * _NSLOTS + b
                pltpu.make_async_copy(
                    table_hbm.at[idx_v.at[k]], buf.at[b], sem_g[b]).wait()
                pltpu.async_copy(
                    buf.at[b], acc_s.at[tgt_v.at[k]], sem_s[b], add=True)
                g = k + 2
                bg = (b + 2) % _NSLOTS

                @pl.when(g < rpw)
                def _():
                    @pl.when(g >= _NSLOTS)
                    def _():
                        pltpu.make_async_copy(
                            buf.at[bg], acc_s.at[tgt_v.at[k]], sem_s[bg]).wait()

                    pltpu.async_copy(
                        table_hbm.at[idx_v.at[g]], buf.at[bg], sem_g[bg])
            return carry

        lax.fori_loop(0, rpw // _NSLOTS, steps, 0)

        for b in range(_NSLOTS):
            pltpu.make_async_copy(
                buf.at[b], acc_s.at[tgt_v.at[0]], sem_s[b]).wait()

        pltpu.sync_copy(acc_s.at[pl.ds(sid * ipw, ipw)],
                        out_hbm.at[pl.ds(wid * ipw, ipw)])

    return body(ids2, tgt2, table)


def _mm_body(scale, r_ref, t_ref, o_ref):
    o_ref[...] = jnp.dot(
        r_ref[...], t_ref[...], preferred_element_type=jnp.float32) * scale


def _matmul_tc(ratio, sums, scale):
    b, n = ratio.shape
    _, d = sums.shape
    bb = 256
    return pl.pallas_call(
        functools.partial(_mm_body, scale),
        grid=(b // bb,),
        in_specs=[
            pl.BlockSpec((bb, n), lambda i: (i, 0)),
            pl.BlockSpec((n, d), lambda i: (0, 0)),
        ],
        out_specs=pl.BlockSpec((bb, d), lambda i: (i, 0)),
        out_shape=jax.ShapeDtypeStruct((b, d), jnp.float32),
    )(ratio, sums)


def kernel(input_ids, input_ratio, embedding):
    n_items, hist = input_ids.shape
    clen = _CHUNK_ITEMS * hist
    info = plsc.get_sparse_core_info()
    nw = info.num_cores * info.num_subcores
    ipw = n_items // nw
    ids2 = input_ids.astype(jnp.int32).reshape(n_items * hist // clen, clen)
    ns = info.num_subcores
    local = jnp.repeat(jnp.arange(ipw, dtype=jnp.int32), hist)
    tgt3 = (local[None, :] + (jnp.arange(ns, dtype=jnp.int32) * ipw)[:, None]
            ).reshape(ns, ipw // _CHUNK_ITEMS, clen)
    sums = _gather_sum_sc(ids2, tgt3, embedding, n_items, hist)
    return _matmul_tc(input_ratio, sums, float(1.0 / hist))
